# baseline (speedup 1.0000x reference)
import jax
import jax.numpy as jnp
from jax import lax
from jax.experimental import pallas as pl
from jax.experimental.pallas import tpu as pltpu
from jax.experimental.pallas import tpu_sc as plsc

_TOKENS = 8192
_DIM = 2048
_EXPERTS = 64
_LANES = 16
_WORKERS = 32
_TPW = _TOKENS // _WORKERS
_GROUPS = _TPW // _LANES


def _routing_body(logits_t_hbm, idx_hbm, w_hbm, lv, idx_v, w_v):
    wid = lax.axis_index("s") * 2 + lax.axis_index("c")
    base = wid * _TPW
    pltpu.sync_copy(logits_t_hbm.at[:, pl.ds(base, _TPW)], lv)
    lanes = lax.iota(jnp.int32, _LANES)
    neg = jnp.full((_LANES,), jnp.finfo(jnp.float32).min, jnp.float32)
    zero = jnp.zeros((_LANES,), jnp.int32)

    for g in range(_GROUPS):
        col = g * _LANES

        def step(e, carry, _col=col):
            m1, i1, m2, i2 = carry
            v = lv[e, pl.ds(_col, _LANES)]
            ev = zero + e
            gt1 = v > m1
            gt2 = v > m2
            m2n = jnp.where(gt1, m1, jnp.where(gt2, v, m2))
            i2n = jnp.where(gt1, i1, jnp.where(gt2, ev, i2))
            m1n = jnp.where(gt1, v, m1)
            i1n = jnp.where(gt1, ev, i1)
            return m1n, i1n, m2n, i2n

        m1, i1, m2, i2 = lax.fori_loop(
            0, _EXPERTS, step, (neg, zero, neg, zero), unroll=8)
        z = jnp.exp(m2 - m1)
        w1 = 1.0 / (1.0 + z)
        w2 = 1.0 - w1
        pos = (col + lanes) * 2
        plsc.store_scatter(idx_v, [pos], i1)
        plsc.store_scatter(idx_v, [pos + 1], i2)
        plsc.store_scatter(w_v, [pos], w1)
        plsc.store_scatter(w_v, [pos + 1], w2)

    pltpu.sync_copy(idx_v, idx_hbm.at[pl.ds(base * 2, _TPW * 2)])
    pltpu.sync_copy(w_v, w_hbm.at[pl.ds(base * 2, _TPW * 2)])


def _route(router_logits):
    logits_t = router_logits.T
    mesh = plsc.VectorSubcoreMesh(core_axis_name="c", subcore_axis_name="s")
    run = pl.kernel(
        _routing_body,
        out_type=[
            jax.ShapeDtypeStruct((_TOKENS * 2,), jnp.int32),
            jax.ShapeDtypeStruct((_TOKENS * 2,), jnp.float32),
        ],
        mesh=mesh,
        scratch_types=[
            pltpu.VMEM((_EXPERTS, _TPW), jnp.float32),
            pltpu.VMEM((_TPW * 2,), jnp.int32),
            pltpu.VMEM((_TPW * 2,), jnp.float32),
        ],
        name="sc_top2_router",
        compiler_params=pltpu.CompilerParams(needs_layout_passes=False),
    )
    return run(logits_t)


_BLK = 512


def _expand_body(x_ref, out_ref):
    v3 = x_ref[...].reshape(_BLK // 8, 8, _DIM)
    sub = lax.broadcasted_iota(jnp.int32, (_BLK // 8, 8, _DIM), 1) // 2
    a = jnp.take_along_axis(v3, sub, axis=1)
    b = jnp.take_along_axis(v3, sub + 4, axis=1)
    out_ref[...] = jnp.stack([a, b], axis=1).reshape(2 * _BLK, _DIM)


def _expand(x):
    return pl.pallas_call(
        _expand_body,
        grid=(_TOKENS // _BLK,),
        in_specs=[pl.BlockSpec((_BLK, _DIM), lambda i: (i, 0))],
        out_specs=pl.BlockSpec((2 * _BLK, _DIM), lambda i: (i, 0)),
        out_shape=jax.ShapeDtypeStruct((_TOKENS * 2, _DIM), jnp.float32),
        compiler_params=pltpu.CompilerParams(
            dimension_semantics=("parallel",)),
        name="tc_expand",
    )(x)


def kernel(x, router_logits):
    expert_indices, expert_weights = _route(router_logits)
    expanded_x = _expand(x)
    return (expanded_x, expert_indices, expert_weights)


# speedup vs baseline: 1.0289x; 1.0289x over previous
import jax
import jax.numpy as jnp
from jax import lax
from jax.experimental import pallas as pl
from jax.experimental.pallas import tpu as pltpu
from jax.experimental.pallas import tpu_sc as plsc

_TOKENS = 8192
_DIM = 2048
_EXPERTS = 64
_LANES = 16
_WORKERS = 32
_TPW = _TOKENS // _WORKERS
_GROUPS = _TPW // _LANES


def _routing_body(logits_t_hbm, idx_hbm, w_hbm, lv, idx_v, w_v):
    wid = lax.axis_index("s") * 2 + lax.axis_index("c")
    base = wid * _TPW
    pltpu.sync_copy(logits_t_hbm.at[:, pl.ds(base, _TPW)], lv)
    lanes = lax.iota(jnp.int32, _LANES)
    neg = jnp.full((_LANES,), jnp.finfo(jnp.float32).min, jnp.float32)
    zero = jnp.zeros((_LANES,), jnp.int32)

    for g in range(_GROUPS):
        col = g * _LANES

        def step(e, carry, _col=col):
            m1, i1, m2, i2 = carry
            v = lv[e, pl.ds(_col, _LANES)]
            ev = zero + e
            gt1 = v > m1
            gt2 = v > m2
            m2n = jnp.where(gt1, m1, jnp.where(gt2, v, m2))
            i2n = jnp.where(gt1, i1, jnp.where(gt2, ev, i2))
            m1n = jnp.where(gt1, v, m1)
            i1n = jnp.where(gt1, ev, i1)
            return m1n, i1n, m2n, i2n

        m1, i1, m2, i2 = lax.fori_loop(
            0, _EXPERTS, step, (neg, zero, neg, zero), unroll=8)
        z = jnp.exp(m2 - m1)
        w1 = 1.0 / (1.0 + z)
        w2 = 1.0 - w1
        pos = (col + lanes) * 2
        plsc.store_scatter(idx_v, [pos], i1)
        plsc.store_scatter(idx_v, [pos + 1], i2)
        plsc.store_scatter(w_v, [pos], w1)
        plsc.store_scatter(w_v, [pos + 1], w2)

    pltpu.sync_copy(idx_v, idx_hbm.at[pl.ds(base * 2, _TPW * 2)])
    pltpu.sync_copy(w_v, w_hbm.at[pl.ds(base * 2, _TPW * 2)])


def _route(router_logits):
    logits_t = router_logits.T
    mesh = plsc.VectorSubcoreMesh(core_axis_name="c", subcore_axis_name="s")
    run = pl.kernel(
        _routing_body,
        out_type=[
            jax.ShapeDtypeStruct((_TOKENS * 2,), jnp.int32),
            jax.ShapeDtypeStruct((_TOKENS * 2,), jnp.float32),
        ],
        mesh=mesh,
        scratch_types=[
            pltpu.VMEM((_EXPERTS, _TPW), jnp.float32),
            pltpu.VMEM((_TPW * 2,), jnp.int32),
            pltpu.VMEM((_TPW * 2,), jnp.float32),
        ],
        name="sc_top2_router",
        compiler_params=pltpu.CompilerParams(needs_layout_passes=False),
    )
    return run(logits_t)


_BLK = 1024


def _expand_body(x_ref, out_ref):
    v3 = x_ref[...].reshape(_BLK // 8, 8, _DIM)
    sub = lax.broadcasted_iota(jnp.int32, (_BLK // 8, 8, _DIM), 1) // 2
    a = jnp.take_along_axis(v3, sub, axis=1)
    b = jnp.take_along_axis(v3, sub + 4, axis=1)
    out_ref[...] = jnp.stack([a, b], axis=1).reshape(2 * _BLK, _DIM)


def _expand(x):
    return pl.pallas_call(
        _expand_body,
        grid=(_TOKENS // _BLK,),
        in_specs=[pl.BlockSpec((_BLK, _DIM), lambda i: (i, 0))],
        out_specs=pl.BlockSpec((2 * _BLK, _DIM), lambda i: (i, 0)),
        out_shape=jax.ShapeDtypeStruct((_TOKENS * 2, _DIM), jnp.float32),
        compiler_params=pltpu.CompilerParams(
            dimension_semantics=("parallel",)),
        name="tc_expand",
    )(x)


def kernel(x, router_logits):
    expert_indices, expert_weights = _route(router_logits)
    expanded_x = _expand(x)
    return (expanded_x, expert_indices, expert_weights)
